# Initial kernel scaffold; baseline (speedup 1.0000x reference)
import jax
import jax.numpy as jnp
from jax.experimental import pallas as pl


def kernel(x, params):
    raise NotImplementedError("write your pallas kernel here")


# speedup vs baseline: 4.4644x; 4.4644x over previous
import functools
import math

import numpy as np
import jax
import jax.numpy as jnp
from jax.experimental import pallas as pl
from jax.experimental.pallas import tpu as pltpu

F32 = jnp.float32


_T, _TP = 29, 32
_N1, _N1P = 30, 32
_N2, _N2P = 10, 16
_NF = 128
_DF, _DFP = 30, 32
_NR, _NRP = 15, 16


def _np_pe():
    pos = np.arange(_T)[:, None].astype(np.float32)
    div = np.exp(np.arange(0, 128, 2).astype(np.float32) * (-math.log(10000.0) / 128))
    pe = np.zeros((_T, 128), np.float32)
    pe[:, 0::2] = np.sin(pos * div)
    pe[:, 1::2] = np.cos(pos * div)
    out = np.zeros((_TP, 128), np.float32)
    out[:_T] = pe
    return out


def _np_dct():
    freqs = np.linspace(0, 7, _T).astype(np.int64)
    t = np.arange(128).astype(np.float32)
    f = np.stack([np.cos(np.pi * fi * (2.0 * t + 1.0) / (2.0 * 128)) for fi in freqs])
    out = np.zeros((_TP, 128), np.float32)
    out[:_T] = f.astype(np.float32)
    return out


def _np_dft():
    eye_t = np.eye(_T)
    R = np.fft.rfft(eye_t, axis=0)
    RR = np.zeros((_NRP, _TP)); RI = np.zeros((_NRP, _TP))
    RR[:_NR, :_T] = R.real
    RI[:_NR, :_T] = R.imag
    eye_f = np.eye(_NR)
    IR = np.zeros((_TP, _NRP)); II = np.zeros((_TP, _NRP))
    IR[:_T, :_NR] = np.fft.irfft(eye_f, n=_T, axis=0)
    II[:_T, :_NR] = np.fft.irfft(1j * eye_f, n=_T, axis=0)
    return (RR.astype(np.float32), RI.astype(np.float32),
            IR.astype(np.float32), II.astype(np.float32))


_PE = _np_pe()
_DCT = _np_dct()
_RR, _RI, _IR, _II = _np_dft()
_I32 = np.eye(_TP, dtype=np.float32)


_HI = jax.lax.Precision.HIGHEST


def _bmm(a, b, ca, cb, prec=None):
    return jax.lax.dot_general(
        a, b, (((ca,), (cb,)), ((0,), (0,))),
        preferred_element_type=F32, precision=prec)


def _mm3(x, w):
    bb, n, f = x.shape
    return (x.reshape(bb * n, f) @ w).reshape(bb, n, w.shape[1])


def _lmm(m2d, x, prec=None):
    bb = x.shape[0]
    mb = jnp.broadcast_to(m2d[None], (bb,) + m2d.shape)
    return _bmm(mb, x, 2, 1, prec=prec)


def _ln(x, g, b):
    m = jnp.mean(x, -1, keepdims=True)
    v = jnp.mean((x - m) ** 2, -1, keepdims=True)
    return (x - m) / jnp.sqrt(v + 1e-5) * g[None] + b[None]


def _dwconv(a, cw):
    z = jnp.zeros_like(a[:, :1])
    xm1 = jnp.concatenate([z, a[:, :-1]], 1)
    xp1 = jnp.concatenate([a[:, 1:], z], 1)
    xp2 = jnp.concatenate([a[:, 2:], z, z], 1)
    return (cw[0][None, None] * xm1 + cw[1][None, None] * a
            + cw[2][None, None] * xp1 + cw[3][None, None] * xp2)


def _cos_hyper(x, wp, bp, k, n_valid):
    bb, n, d = x.shape
    inv = 1.0 / (jnp.sqrt(jnp.sum(x * x, -1, keepdims=True)) + 1e-8)
    xn = x * inv
    sim = _bmm(xn, xn, 2, 2)
    col = jax.lax.broadcasted_iota(jnp.int32, (bb, n, n), 2)
    neg = jnp.float32(-3e38)
    if n_valid < n:
        sim = jnp.where(col < n_valid, sim, neg)
    memb = jnp.zeros_like(sim)
    s = sim
    for _ in range(k):
        mx = jnp.max(s, axis=2, keepdims=True)
        ismx = s >= mx
        first = jnp.min(jnp.where(ismx, col, n), axis=2, keepdims=True)
        sel = col == first
        memb = jnp.where(sel, 1.0, memb)
        s = jnp.where(sel, neg, s)
    if n_valid < n:
        row = jax.lax.broadcasted_iota(jnp.int32, (bb, n, n), 1)
        memb = jnp.where(row < n_valid, memb, 0.0)
    ones_col = jnp.ones((bb, n, 1), F32)
    dv = _bmm(memb, ones_col, 1, 1)
    theta = 1.0 / jnp.sqrt(jnp.maximum(dv, 1.0))
    m_feat = x * theta
    e_agg = _bmm(memb, m_feat, 2, 1) * (1.0 / k)
    n2 = _bmm(memb, e_agg, 1, 1)
    feats = _mm3(n2 * theta, wp) + bp[None]
    return feats, memb


def _gat(x, mask, wh, ash, adh, bp, heads):
    bb, n, f = x.shape
    out = None
    ones_col = jnp.ones((bb, n, 1), F32)
    for h in range(heads):
        hh = _mm3(x, wh[h])
        asr = jnp.sum(hh * ash[h][None, None], -1, keepdims=True)
        ads = jnp.sum(hh * adh[h][None, None], -1, keepdims=True)
        asr_t = _bmm(ones_col, asr, 2, 2, prec=_HI)
        e = ads + asr_t
        e = jnp.where(e >= 0, e, 0.2 * e)
        e = jnp.where(mask[None], e, -1e9)
        mx = jnp.max(e, -1, keepdims=True)
        p = jnp.exp(e - mx)
        attn = p / jnp.sum(p, -1, keepdims=True)
        oh = _bmm(attn, hh, 2, 1)
        out = oh if out is None else out + oh
    return out + bp[None]


def _eye_mask(a_sum):
    n = a_sum.shape[0]
    r = jax.lax.broadcasted_iota(jnp.int32, (n, n), 0)
    c = jax.lax.broadcasted_iota(jnp.int32, (n, n), 1)
    return (a_sum > 0) | (r == c)


def _elu(x):
    return jnp.where(x > 0, x, jnp.exp(jnp.minimum(x, 0.0)) - 1.0)


def _k1_body(xb, pe, ln1g, ln1b, inp, cw, outp, ln2g, ln2b, wr, wi,
             w1, b1, w2, b2, rr, ri, irm, iim, filt, fca1, fca2, ffc1, ffc2,
             lng, lnb, hg1w, hg1b, i32, hg11w, hg11b, hg22w, hg22b,
             h1_o, xf2_o, a1_o, af2_o):
    x1 = xb[...] + pe[...][None]
    for i in range(2):
        h = _ln(x1, ln1g[i], ln1b[i])
        a = _mm3(h, inp[i])
        a1 = jax.nn.silu(_dwconv(a[:, :, :128], cw[i]))
        a2 = jax.nn.silu(a[:, :, 128:])
        x1 = x1 + _mm3(a1 * a2, outp[i])
        hh = _ln(x1, ln2g[i], ln2b[i])
        fr = _lmm(rr[...], hh, prec=_HI)
        fi = _lmm(ri[...], hh, prec=_HI)
        gr = fr * wr[i][None] - fi * wi[i][None]
        gi = fr * wi[i][None] + fi * wr[i][None]
        h2 = _lmm(irm[...], gr, prec=_HI) + _lmm(iim[...], gi, prec=_HI)
        h2 = _mm3(jax.nn.gelu(_mm3(h2, w1[i]) + b1[i][None]), w2[i]) + b2[i][None]
        x1 = x1 + h2
    y = _mm3(jax.nn.relu(_mm3(x1, fca1[...])), fca2[...])
    pooled = jnp.sum(y * filt[...][None], -1)
    att = jax.nn.sigmoid(jax.nn.relu(pooled @ ffc1[...]) @ ffc2[...])
    y = y * att[:, :, None]
    h = jax.nn.relu(y + x1)
    h = _ln(h, lng[...], lnb[...])
    mr = jnp.sum(h, 1, keepdims=True) * (1.0 / _T)
    riota = jax.lax.broadcasted_iota(jnp.int32, h.shape, 1)
    hcat = jnp.where(riota == _T, mr, h)
    h1, m1 = _cos_hyper(hcat, hg1w[...], hg1b[...], 5, _N1)
    h1_o[...] = jax.nn.relu(h1)
    c1 = _bmm(m1, m1, 1, 1)
    xf = _bmm(hcat, jnp.broadcast_to(i32[...][None], (hcat.shape[0], _TP, _TP)),
              1, 1, prec=_HI)
    xf1, _ = _cos_hyper(xf, hg11w[...], hg11b[...], 10, _NF)
    xf1 = jax.nn.relu(xf1)
    xf2, mf2 = _cos_hyper(xf1, hg22w[...], hg22b[...], 10, _NF)
    xf2_o[...] = jax.nn.relu(xf2)
    cf2 = _bmm(mf2, mf2, 1, 1)

    @pl.when(pl.program_id(0) == 0)
    def _():
        a1_o[...] = jnp.zeros_like(a1_o)
        af2_o[...] = jnp.zeros_like(af2_o)

    a1_o[...] += jnp.sum(c1, 0)
    af2_o[...] += jnp.sum(cf2, 0)


def _k2_body(h1b, xf2b, a1s, af2s, g1w, g1as, g1ad, g1b, g2w, g2as, g2ad, g2b,
             tw, hg2w, hg2b, g33w, g33as, g33ad, g33b, g44w, g44as, g44ad, g44b,
             h2_o, outf_o, a2_o):
    mask1 = _eye_mask(a1s[...])
    g1 = _elu(_gat(h1b[...], mask1, g1w[...], g1as[...], g1ad[...], g1b[...], 4))
    g2 = _elu(_gat(g1, mask1, g2w[...], g2as[...], g2ad[...], g2b[...], 1))
    out10 = _lmm(tw[...], g2, prec=_HI)
    h2, m2 = _cos_hyper(out10, hg2w[...], hg2b[...], 5, _N2)
    h2_o[...] = jax.nn.relu(h2)
    c2 = _bmm(m2, m2, 1, 1)
    maskf = _eye_mask(af2s[...])
    f1 = _elu(_gat(xf2b[...], maskf, g33w[...], g33as[...], g33ad[...], g33b[...], 2))
    f2 = _elu(_gat(f1, maskf, g44w[...], g44as[...], g44ad[...], g44b[...], 1))
    outf_o[...] = jnp.sum(f2, 1) * (1.0 / _NF)

    @pl.when(pl.program_id(0) == 0)
    def _():
        a2_o[...] = jnp.zeros_like(a2_o)

    a2_o[...] += jnp.sum(c2, 0)


def _k3_body(h2b, outfb, a2s, g3w, g3as, g3ad, g3b, g4w, g4as, g4ad, g4b,
             fc1a, fc1b, fc1bias, fc2w, fc2b, out_o):
    mask2 = _eye_mask(a2s[...])
    g3 = _elu(_gat(h2b[...], mask2, g3w[...], g3as[...], g3ad[...], g3b[...], 4))
    g4 = _elu(_gat(g3, mask2, g4w[...], g4as[...], g4ad[...], g4b[...], 1))
    out_t = jnp.sum(g4, 1) * (1.0 / _N2)
    z = jax.nn.relu(out_t @ fc1a[...] + outfb[...] @ fc1b[...] + fc1bias[...])
    out_o[...] = z @ fc2w[...] + fc2b[...]


def _pad2(a, r, c):
    return jnp.pad(a, ((0, r - a.shape[0]), (0, c - a.shape[1])))


def _gat_prep(p, name, heads, od, fin_pad, hod_pad):
    w, a_s, a_d, b = p[name + '_W'], p[name + '_as'], p[name + '_ad'], p[name + '_b']
    fin = w.shape[0]
    wh = jnp.zeros((heads, fin_pad, hod_pad), F32)
    ash = jnp.zeros((heads, hod_pad), F32)
    adh = jnp.zeros((heads, hod_pad), F32)
    for h in range(heads):
        wh = wh.at[h, :fin, h * od:(h + 1) * od].set(w[:, h * od:(h + 1) * od])
        ash = ash.at[h, h * od:(h + 1) * od].set(a_s[h])
        adh = adh.at[h, h * od:(h + 1) * od].set(a_d[h])
    bp = _pad2(b[None], 1, hod_pad)
    return wh, ash, adh, bp


def _cspec(shape):
    nd = len(shape)
    return pl.BlockSpec(shape, lambda i, _n=nd: (0,) * _n)


def _bspec(shape):
    nd = len(shape)
    return pl.BlockSpec(shape, lambda i, _n=nd: (i,) + (0,) * (_n - 1))


def kernel(x, params):
    p = params
    B = x.shape[0]
    xp = jnp.pad(x[:, 0], ((0, 0), (0, _TP - _T), (0, 0)))

    def stk(fn):
        return jnp.stack([fn('b0_'), fn('b1_')])

    ln1g = stk(lambda q: p[q + 'ln1_g'][None])
    ln1b = stk(lambda q: p[q + 'ln1_b'][None])
    inp = stk(lambda q: p[q + 'in_proj'])
    cwm = stk(lambda q: p[q + 'conv_w'][:, 0, :].T)
    outp = stk(lambda q: p[q + 'out_proj'])
    ln2g = stk(lambda q: p[q + 'ln2_g'][None])
    ln2b = stk(lambda q: p[q + 'ln2_b'][None])
    wr = stk(lambda q: _pad2(p[q + 'fft_wr'], _NRP, 128))
    wi = stk(lambda q: _pad2(p[q + 'fft_wi'], _NRP, 128))
    w1 = stk(lambda q: _pad2(p[q + 'mlp_w1'], 128, 128))
    b1 = stk(lambda q: _pad2(p[q + 'mlp_b1'][None], 1, 128))
    w2 = stk(lambda q: _pad2(p[q + 'mlp_w2'], 128, 128))
    b2 = stk(lambda q: p[q + 'mlp_b2'][None])

    pe = jnp.asarray(_PE)
    filt = jnp.asarray(_DCT)
    rr, ri, irm, iim = (jnp.asarray(a) for a in (_RR, _RI, _IR, _II))
    i32 = jnp.asarray(_I32)
    ffc1 = _pad2(p['fca_fc1'], _TP, 8)
    ffc2 = _pad2(p['fca_fc2'], 8, _TP)
    lng, lnb = p['ln_g'][None], p['ln_b'][None]
    hg1w, hg1b = p['hg1_W'], p['hg1_b'][None]
    hg11w, hg11b = _pad2(p['hg11_W'], _DFP, _DFP), _pad2(p['hg11_b'][None], 1, _DFP)
    hg22w, hg22b = _pad2(p['hg22_W'], _DFP, _DFP), _pad2(p['hg22_b'][None], 1, _DFP)

    bb1 = 8
    k1_in = (xp, pe, ln1g, ln1b, inp, cwm, outp, ln2g, ln2b, wr, wi,
             w1, b1, w2, b2, rr, ri, irm, iim, filt, p['fca_conv1'],
             p['fca_conv2'], ffc1, ffc2, lng, lnb, hg1w, hg1b, i32,
             hg11w, hg11b, hg22w, hg22b)
    k1_specs = [_bspec((bb1, _TP, 128))] + [_cspec(a.shape) for a in k1_in[1:]]
    h1, xf2, a1s, af2s = pl.pallas_call(
        _k1_body,
        grid=(B // bb1,),
        in_specs=k1_specs,
        out_specs=[_bspec((bb1, _TP, 128)), _bspec((bb1, _NF, _DFP)),
                   _cspec((_N1P, _N1P)), _cspec((_NF, _NF))],
        out_shape=[jax.ShapeDtypeStruct((B, _TP, 128), F32),
                   jax.ShapeDtypeStruct((B, _NF, _DFP), F32),
                   jax.ShapeDtypeStruct((_N1P, _N1P), F32),
                   jax.ShapeDtypeStruct((_NF, _NF), F32)],
        compiler_params=pltpu.CompilerParams(dimension_semantics=("arbitrary",)),
    )(*k1_in)

    g1w, g1as, g1ad, g1b = _gat_prep(p, 'gat1', 4, 32, 128, 128)
    g2w, g2as, g2ad, g2b = _gat_prep(p, 'gat2', 1, 128, 128, 128)
    g33w, g33as, g33ad, g33b = _gat_prep(p, 'gat33', 2, 15, _DFP, _DFP)
    g44w, g44as, g44ad, g44b = _gat_prep(p, 'gat44', 1, 30, _DFP, _DFP)
    tw = jnp.zeros((_N2P, _N1P), F32).at[
        jnp.arange(10)[:, None], 3 * jnp.arange(10)[:, None] + jnp.arange(3)[None, :]
    ].set(p['time_weight'])
    hg2w, hg2b = p['hg2_W'], p['hg2_b'][None]

    bb2 = 8
    k2_in = (h1, xf2, a1s, af2s, g1w, g1as, g1ad, g1b, g2w, g2as, g2ad, g2b,
             tw, hg2w, hg2b, g33w, g33as, g33ad, g33b, g44w, g44as, g44ad, g44b)
    k2_specs = ([_bspec((bb2, _TP, 128)), _bspec((bb2, _NF, _DFP))]
                + [_cspec(a.shape) for a in k2_in[2:]])
    h2, outf, a2s = pl.pallas_call(
        _k2_body,
        grid=(B // bb2,),
        in_specs=k2_specs,
        out_specs=[_bspec((bb2, _N2P, 128)), _bspec((bb2, _DFP)),
                   _cspec((_N2P, _N2P))],
        out_shape=[jax.ShapeDtypeStruct((B, _N2P, 128), F32),
                   jax.ShapeDtypeStruct((B, _DFP), F32),
                   jax.ShapeDtypeStruct((_N2P, _N2P), F32)],
        compiler_params=pltpu.CompilerParams(dimension_semantics=("arbitrary",)),
    )(*k2_in)

    g3w, g3as, g3ad, g3b = _gat_prep(p, 'gat3', 4, 32, 128, 128)
    g4w, g4as, g4ad, g4b = _gat_prep(p, 'gat4', 1, 128, 128, 128)
    fc1a = p['fc1_w'][:128]
    fc1b = _pad2(p['fc1_w'][128:], _DFP, 128)
    fc1bias = p['fc1_b'][None]
    fc2w = _pad2(p['fc2_w'], 128, 8)
    fc2b = _pad2(p['fc2_b'][None], 1, 8)

    bb3 = 64
    k3_in = (h2, outf, a2s, g3w, g3as, g3ad, g3b, g4w, g4as, g4ad, g4b,
             fc1a, fc1b, fc1bias, fc2w, fc2b)
    k3_specs = ([_bspec((bb3, _N2P, 128)), _bspec((bb3, _DFP))]
                + [_cspec(a.shape) for a in k3_in[2:]])
    out = pl.pallas_call(
        _k3_body,
        grid=(B // bb3,),
        in_specs=k3_specs,
        out_specs=_bspec((bb3, 8)),
        out_shape=jax.ShapeDtypeStruct((B, 8), F32),
        compiler_params=pltpu.CompilerParams(dimension_semantics=("arbitrary",)),
    )(*k3_in)
    return out[:, :4]
